# Initial kernel scaffold; baseline (speedup 1.0000x reference)
import jax
import jax.numpy as jnp
from jax.experimental import pallas as pl


def kernel(x, W_gate, W1, b1, W2, b2):
    raise NotImplementedError("write your pallas kernel here")


# speedup vs baseline: 1.7270x; 1.7270x over previous
import functools

import jax
import jax.numpy as jnp
from jax import lax
from jax.experimental import pallas as pl
from jax.experimental.pallas import tpu as pltpu
from jax.experimental.pallas import tpu_sc as plsc

T, D, H, E, K = 2048, 1024, 512, 8, 2
BLK = 128
NB = T * K // BLK + E
R = NB * BLK
NW = 32
PAIRS_PER_W = T * K // NW
CH = 64
NSH = PAIRS_PER_W // CH
TPW = T // NW
TPS = TPW // NSH


def _router_body(x_ref, wg_ref, dest_ref, gexp_ref, be_ref):
    x = x_ref[...]
    logits = jnp.dot(x, wg_ref[...], preferred_element_type=jnp.float32)
    m = jnp.max(logits, axis=-1, keepdims=True)
    ex = jnp.exp(logits - m)
    p = ex / jnp.sum(ex, axis=-1, keepdims=True)
    lane = lax.broadcasted_iota(jnp.int32, (T, E), 1)
    v1 = jnp.max(p, axis=-1, keepdims=True)
    i1 = jnp.min(jnp.where(p == v1, lane, E), axis=-1, keepdims=True)
    p2 = jnp.where(lane == i1, -jnp.inf, p)
    v2 = jnp.max(p2, axis=-1, keepdims=True)
    i2 = jnp.min(jnp.where(p2 == v2, lane, E), axis=-1, keepdims=True)
    oh1 = (lane == i1).astype(jnp.float32)
    oh2 = (lane == i2).astype(jnp.float32)
    c = oh1 + oh2
    rowi = lax.broadcasted_iota(jnp.int32, (T, T), 0)
    coli = lax.broadcasted_iota(jnp.int32, (T, T), 1)
    tri = (coli < rowi).astype(jnp.float32)
    cum = jnp.dot(tri, c, preferred_element_type=jnp.float32)
    tot = jnp.sum(c, axis=0, keepdims=True)
    pc = jnp.floor((tot + (BLK - 1)) * (1.0 / BLK)) * BLK
    ei = lax.broadcasted_iota(jnp.int32, (E, E), 0)
    ej = lax.broadcasted_iota(jnp.int32, (E, E), 1)
    triE = (ei < ej).astype(jnp.float32)
    base = jnp.dot(pc, triE, preferred_element_type=jnp.float32)
    ends = base + pc
    base_t = jnp.broadcast_to(base, (T, E)) + cum
    d1 = jnp.sum(oh1 * base_t, axis=-1, keepdims=True)
    d2 = jnp.sum(oh2 * base_t, axis=-1, keepdims=True)
    dest_ref[...] = jnp.concatenate([d1, d2], axis=1).astype(jnp.int32)
    gexp_ref[...] = jnp.concatenate(
        [jnp.broadcast_to(v1, (T, 16)), jnp.broadcast_to(v2, (T, 16))], axis=1)
    bv = lax.broadcasted_iota(jnp.int32, (64, E), 0).astype(jnp.float32) * BLK
    nend = jnp.sum((bv >= jnp.broadcast_to(ends, (64, E))).astype(jnp.float32),
                   axis=-1, keepdims=True)
    be = jnp.minimum(nend, float(E - 1))
    be_ref[...] = jnp.broadcast_to(be, (64, 128)).astype(jnp.int32)


def _router(x, wg):
    return pl.pallas_call(
        _router_body,
        out_shape=(
            jax.ShapeDtypeStruct((T, K), jnp.int32),
            jax.ShapeDtypeStruct((T, 2 * 16), jnp.float32),
            jax.ShapeDtypeStruct((64, 128), jnp.int32),
        ),
    )(x, wg)


def _dispatch(x, toks_r, dest_r):
    mesh = plsc.VectorSubcoreMesh(core_axis_name="c", subcore_axis_name="s")

    @functools.partial(
        pl.kernel, mesh=mesh,
        out_type=jax.ShapeDtypeStruct((R, D), jnp.float32),
        scratch_types=[
            pltpu.VMEM((NSH, CH), jnp.int32),
            pltpu.VMEM((NSH, CH), jnp.int32),
            pltpu.VMEM((CH, D), jnp.float32),
            pltpu.SemaphoreType.DMA,
        ],
    )
    def k(x_hbm, toks_hbm, dest_hbm, xs_hbm, tok_v, dst_v, rows_v, sem):
        wid = lax.axis_index("s") * 2 + lax.axis_index("c")
        pltpu.sync_copy(toks_hbm.at[wid], tok_v)
        pltpu.sync_copy(dest_hbm.at[wid], dst_v)
        for s in range(NSH):
            pltpu.async_copy(x_hbm.at[tok_v.at[s]], rows_v, sem).wait()
            pltpu.async_copy(rows_v, xs_hbm.at[dst_v.at[s]], sem).wait()

    return k(x, toks_r, dest_r)


def _ffn_body(be_ref, xs_ref, w1_ref, b1_ref, w2_ref, b2_ref, ys_ref):
    del be_ref
    xb = xs_ref[...]
    h = jnp.dot(xb, w1_ref[0], preferred_element_type=jnp.float32) + b1_ref[0]
    h = 0.5 * h * (1.0 + lax.erf(h * 0.7071067811865476))
    y = jnp.dot(h, w2_ref[0], preferred_element_type=jnp.float32) + b2_ref[0]
    ys_ref[...] = y


def _ffn(be, xs, W1, b1, W2, b2):
    grid_spec = pltpu.PrefetchScalarGridSpec(
        num_scalar_prefetch=1,
        grid=(NB,),
        in_specs=[
            pl.BlockSpec((BLK, D), lambda b, be_s: (b, 0)),
            pl.BlockSpec((1, D, H), lambda b, be_s: (be_s[b], 0, 0)),
            pl.BlockSpec((1, 1, H), lambda b, be_s: (be_s[b], 0, 0)),
            pl.BlockSpec((1, H, D), lambda b, be_s: (be_s[b], 0, 0)),
            pl.BlockSpec((1, 1, D), lambda b, be_s: (be_s[b], 0, 0)),
        ],
        out_specs=pl.BlockSpec((BLK, D), lambda b, be_s: (b, 0)),
    )
    return pl.pallas_call(
        _ffn_body,
        grid_spec=grid_spec,
        out_shape=jax.ShapeDtypeStruct((R, D), jnp.float32),
    )(be, xs, W1, b1.reshape(E, 1, H), W2, b2.reshape(E, 1, D))


def _combine(ys, dest_r, gexp_flat):
    mesh = plsc.VectorSubcoreMesh(core_axis_name="c", subcore_axis_name="s")

    @functools.partial(
        pl.kernel, mesh=mesh,
        out_type=jax.ShapeDtypeStruct((T, D), jnp.float32),
        scratch_types=[
            pltpu.VMEM((NSH, CH), jnp.int32),
            pltpu.VMEM((TPW * 32,), jnp.float32),
            pltpu.VMEM((CH, D), jnp.float32),
            pltpu.VMEM((TPS, D), jnp.float32),
            pltpu.SemaphoreType.DMA,
        ],
    )
    def k(ys_hbm, dest_hbm, gexp_hbm, out_hbm, dst_v, g_v, rows_v, out_v, sem):
        wid = lax.axis_index("s") * 2 + lax.axis_index("c")
        pltpu.sync_copy(dest_hbm.at[wid], dst_v)
        pltpu.sync_copy(gexp_hbm.at[pl.ds(wid * (TPW * 32), TPW * 32)], g_v)
        for s in range(NSH):
            pltpu.async_copy(ys_hbm.at[dst_v.at[s]], rows_v, sem).wait()

            def body(j, _, s=s):
                gbase = (s * TPS + j) * 32
                g0 = g_v[pl.ds(gbase, 16)]
                g1 = g_v[pl.ds(gbase + 16, 16)]

                def inner(cidx, _):
                    sl = pl.ds(cidx * 16, 16)
                    r0 = rows_v[2 * j, sl]
                    r1 = rows_v[2 * j + 1, sl]
                    out_v[j, sl] = g0 * r0 + g1 * r1
                    return 0

                lax.fori_loop(0, D // 16, inner, 0)
                return 0

            lax.fori_loop(0, TPS, body, 0)
            pltpu.sync_copy(out_v, out_hbm.at[pl.ds(wid * TPW + s * TPS, TPS)])

    return k(ys, dest_r, gexp_flat)


def kernel(x, W_gate, W1, b1, W2, b2):
    dest, gexp, be_pad = _router(x, W_gate)
    be = be_pad[:NB, 0]
    dest_r = dest.reshape(NW, NSH, CH)
    toks_r = (jnp.arange(T * K, dtype=jnp.int32) // K).reshape(NW, NSH, CH)
    xs = _dispatch(x, toks_r, dest_r)
    ys = _ffn(be, xs, W1, b1, W2, b2)
    return _combine(ys, dest_r, gexp.reshape(-1))


# speedup vs baseline: 1.7648x; 1.0219x over previous
import functools

import jax
import jax.numpy as jnp
from jax import lax
from jax.experimental import pallas as pl
from jax.experimental.pallas import tpu as pltpu
from jax.experimental.pallas import tpu_sc as plsc

T, D, H, E, K = 2048, 1024, 512, 8, 2
BLK = 128
NB = T * K // BLK + E
R = NB * BLK
NW = 32
PAIRS_PER_W = T * K // NW
CH = 32
NSH = PAIRS_PER_W // CH
TPW = T // NW
TPS = CH // K


def _router_body(x_ref, wg_ref, dest_ref, gexp_ref, be_ref):
    x = x_ref[...]
    logits = jnp.dot(x, wg_ref[...], preferred_element_type=jnp.float32)
    m = jnp.max(logits, axis=-1, keepdims=True)
    ex = jnp.exp(logits - m)
    p = ex / jnp.sum(ex, axis=-1, keepdims=True)
    lane = lax.broadcasted_iota(jnp.int32, (T, E), 1)
    v1 = jnp.max(p, axis=-1, keepdims=True)
    i1 = jnp.min(jnp.where(p == v1, lane, E), axis=-1, keepdims=True)
    p2 = jnp.where(lane == i1, -jnp.inf, p)
    v2 = jnp.max(p2, axis=-1, keepdims=True)
    i2 = jnp.min(jnp.where(p2 == v2, lane, E), axis=-1, keepdims=True)
    oh1 = (lane == i1).astype(jnp.float32)
    oh2 = (lane == i2).astype(jnp.float32)
    c = oh1 + oh2
    rowi = lax.broadcasted_iota(jnp.int32, (T, T), 0)
    coli = lax.broadcasted_iota(jnp.int32, (T, T), 1)
    tri = (coli < rowi).astype(jnp.bfloat16)
    cum = jnp.dot(tri, c.astype(jnp.bfloat16),
                  preferred_element_type=jnp.float32)
    tot = jnp.sum(c, axis=0, keepdims=True)
    pc = jnp.floor((tot + (BLK - 1)) * (1.0 / BLK)) * BLK
    ei = lax.broadcasted_iota(jnp.int32, (E, E), 0)
    ej = lax.broadcasted_iota(jnp.int32, (E, E), 1)
    triE = (ei < ej).astype(jnp.float32)
    base = jnp.dot(pc, triE, preferred_element_type=jnp.float32)
    ends = base + pc
    base_t = jnp.broadcast_to(base, (T, E)) + cum
    d1 = jnp.sum(oh1 * base_t, axis=-1, keepdims=True)
    d2 = jnp.sum(oh2 * base_t, axis=-1, keepdims=True)
    dest_ref[...] = jnp.concatenate([d1, d2], axis=1).astype(jnp.int32)
    gexp_ref[...] = jnp.concatenate(
        [jnp.broadcast_to(v1, (T, 16)), jnp.broadcast_to(v2, (T, 16))], axis=1)
    bv = lax.broadcasted_iota(jnp.int32, (64, E), 0).astype(jnp.float32) * BLK
    nend = jnp.sum((bv >= jnp.broadcast_to(ends, (64, E))).astype(jnp.float32),
                   axis=-1, keepdims=True)
    be = jnp.minimum(nend, float(E - 1))
    be_ref[...] = jnp.broadcast_to(be, (64, 128)).astype(jnp.int32)


def _router(x, wg):
    return pl.pallas_call(
        _router_body,
        out_shape=(
            jax.ShapeDtypeStruct((T, K), jnp.int32),
            jax.ShapeDtypeStruct((T, 2 * 16), jnp.float32),
            jax.ShapeDtypeStruct((64, 128), jnp.int32),
        ),
    )(x, wg)


def _dispatch(x, toks_r, dest_r):
    mesh = plsc.VectorSubcoreMesh(core_axis_name="c", subcore_axis_name="s")

    @functools.partial(
        pl.kernel, mesh=mesh,
        out_type=jax.ShapeDtypeStruct((R, D), jnp.float32),
        scratch_types=[
            pltpu.VMEM((NSH, CH), jnp.int32),
            pltpu.VMEM((NSH, CH), jnp.int32),
            pltpu.VMEM((CH, D), jnp.float32),
            pltpu.VMEM((CH, D), jnp.float32),
            pltpu.SemaphoreType.DMA,
            pltpu.SemaphoreType.DMA,
        ],
    )
    def k(x_hbm, toks_hbm, dest_hbm, xs_hbm, tok_v, dst_v, rows0, rows1,
          gsem, ssem):
        wid = lax.axis_index("s") * 2 + lax.axis_index("c")
        pltpu.sync_copy(toks_hbm.at[wid], tok_v)
        pltpu.sync_copy(dest_hbm.at[wid], dst_v)
        bufs = (rows0, rows1)
        gops = [None] * NSH
        sops = [None] * NSH
        gops[0] = pltpu.async_copy(x_hbm.at[tok_v.at[0]], rows0, gsem)
        gops[1] = pltpu.async_copy(x_hbm.at[tok_v.at[1]], rows1, gsem)
        for s in range(NSH):
            gops[s].wait()
            sops[s] = pltpu.async_copy(bufs[s % 2], xs_hbm.at[dst_v.at[s]],
                                       ssem)
            if s + 2 < NSH:
                sops[s].wait()
                gops[s + 2] = pltpu.async_copy(
                    x_hbm.at[tok_v.at[s + 2]], bufs[s % 2], gsem)
        sops[NSH - 2].wait()
        sops[NSH - 1].wait()

    return k(x, toks_r, dest_r)


def _ffn_body(be_ref, xs_ref, w1_ref, b1_ref, w2_ref, b2_ref, ys_ref):
    del be_ref
    xb = xs_ref[...].astype(jnp.bfloat16)
    h = jnp.dot(xb, w1_ref[0].astype(jnp.bfloat16),
                preferred_element_type=jnp.float32) + b1_ref[0]
    h = 0.5 * h * (1.0 + lax.erf(h * 0.7071067811865476))
    y = jnp.dot(h.astype(jnp.bfloat16), w2_ref[0].astype(jnp.bfloat16),
                preferred_element_type=jnp.float32) + b2_ref[0]
    ys_ref[...] = y


def _ffn(be, xs, W1, b1, W2, b2):
    grid_spec = pltpu.PrefetchScalarGridSpec(
        num_scalar_prefetch=1,
        grid=(NB,),
        in_specs=[
            pl.BlockSpec((BLK, D), lambda b, be_s: (b, 0)),
            pl.BlockSpec((1, D, H), lambda b, be_s: (be_s[b], 0, 0)),
            pl.BlockSpec((1, 1, H), lambda b, be_s: (be_s[b], 0, 0)),
            pl.BlockSpec((1, H, D), lambda b, be_s: (be_s[b], 0, 0)),
            pl.BlockSpec((1, 1, D), lambda b, be_s: (be_s[b], 0, 0)),
        ],
        out_specs=pl.BlockSpec((BLK, D), lambda b, be_s: (b, 0)),
    )
    return pl.pallas_call(
        _ffn_body,
        grid_spec=grid_spec,
        out_shape=jax.ShapeDtypeStruct((R, D), jnp.float32),
    )(be, xs, W1, b1.reshape(E, 1, H), W2, b2.reshape(E, 1, D))


def _combine(ys, dest_r, gexp_flat):
    mesh = plsc.VectorSubcoreMesh(core_axis_name="c", subcore_axis_name="s")

    @functools.partial(
        pl.kernel, mesh=mesh,
        out_type=jax.ShapeDtypeStruct((T, D), jnp.float32),
        scratch_types=[
            pltpu.VMEM((NSH, CH), jnp.int32),
            pltpu.VMEM((TPW * 32,), jnp.float32),
            pltpu.VMEM((CH, D), jnp.float32),
            pltpu.VMEM((CH, D), jnp.float32),
            pltpu.VMEM((TPS, D), jnp.float32),
            pltpu.SemaphoreType.DMA,
        ],
    )
    def k(ys_hbm, dest_hbm, gexp_hbm, out_hbm, dst_v, g_v, rows0, rows1,
          out_v, sem):
        wid = lax.axis_index("s") * 2 + lax.axis_index("c")
        pltpu.sync_copy(dest_hbm.at[wid], dst_v)
        pltpu.sync_copy(gexp_hbm.at[pl.ds(wid * (TPW * 32), TPW * 32)], g_v)
        bufs = (rows0, rows1)
        gops = [None] * NSH
        gops[0] = pltpu.async_copy(ys_hbm.at[dst_v.at[0]], rows0, sem)
        gops[1] = pltpu.async_copy(ys_hbm.at[dst_v.at[1]], rows1, sem)
        for s in range(NSH):
            gops[s].wait()
            rows_v = bufs[s % 2]

            def body(j, _, s=s, rows_v=rows_v):
                gbase = (s * TPS + j) * 32
                g0 = g_v[pl.ds(gbase, 16)]
                g1 = g_v[pl.ds(gbase + 16, 16)]
                for cidx in range(D // 16):
                    sl = pl.ds(cidx * 16, 16)
                    out_v[j, sl] = g0 * rows_v[2 * j, sl] + g1 * rows_v[2 * j + 1, sl]
                return 0

            lax.fori_loop(0, TPS, body, 0)
            pltpu.sync_copy(out_v, out_hbm.at[pl.ds(wid * TPW + s * TPS, TPS)])
            if s + 2 < NSH:
                gops[s + 2] = pltpu.async_copy(
                    ys_hbm.at[dst_v.at[s + 2]], rows_v, sem)

    return k(ys, dest_r, gexp_flat)


def kernel(x, W_gate, W1, b1, W2, b2):
    dest, gexp, be_pad = _router(x, W_gate)
    be = be_pad[:NB, 0]
    dest_r = dest.reshape(NW, NSH, CH)
    toks_r = (jnp.arange(T * K, dtype=jnp.int32) // K).reshape(NW, NSH, CH)
    xs = _dispatch(x, toks_r, dest_r)
    ys = _ffn(be, xs, W1, b1, W2, b2)
    return _combine(ys, dest_r, gexp.reshape(-1))
